# baseline (speedup 1.0000x reference)
import jax, jax.numpy as jnp
from jax.experimental import pallas as pl
from jax.experimental.pallas import tpu as pltpu

_K = 8
_T_BLOCK = 1024


def _router_body(x_ref, w_hbm, bias_ref, out_ref, w_vmem, sem):
    @pl.when(pl.program_id(0) == 0)
    def _stage_w():
        copy = pltpu.make_async_copy(w_hbm, w_vmem, sem)
        copy.start()
        copy.wait()

    logits = jnp.dot(x_ref[...], w_vmem[...], preferred_element_type=jnp.float32)
    logits = logits + bias_ref[...]
    rowmax = jnp.max(logits, axis=-1, keepdims=True)
    work = jnp.where(logits == rowmax, -jnp.inf, logits)
    for _ in range(_K - 2):
        m = jnp.max(work, axis=-1, keepdims=True)
        work = jnp.where(work == m, -jnp.inf, work)
    thresh = jnp.max(work, axis=-1, keepdims=True)
    ew = jnp.where(logits >= thresh, jnp.exp(logits - rowmax), 0.0)
    out_ref[...] = ew / jnp.sum(ew, axis=-1, keepdims=True)


def kernel(x, w_gate, b_gate, expert_biases):
    t_dim, d_dim = x.shape
    e_dim = w_gate.shape[1]
    bias = (b_gate + expert_biases).reshape(1, e_dim).astype(jnp.float32)
    return pl.pallas_call(
        _router_body,
        grid=(t_dim // _T_BLOCK,),
        in_specs=[
            pl.BlockSpec((_T_BLOCK, d_dim), lambda i: (i, 0)),
            pl.BlockSpec(memory_space=pl.ANY),
            pl.BlockSpec((1, e_dim), lambda i: (0, 0)),
        ],
        out_specs=pl.BlockSpec((_T_BLOCK, e_dim), lambda i: (i, 0)),
        out_shape=jax.ShapeDtypeStruct((t_dim, e_dim), jnp.float32),
        scratch_shapes=[
            pltpu.VMEM((d_dim, e_dim), jnp.float32),
            pltpu.SemaphoreType.DMA,
        ],
        compiler_params=pltpu.CompilerParams(
            dimension_semantics=("arbitrary",),
        ),
    )(x, w_gate, bias)


# speedup vs baseline: 1.0447x; 1.0447x over previous
import jax
import jax.numpy as jnp
from jax.experimental import pallas as pl
from jax.experimental.pallas import tpu as pltpu

_K = 8
_T_BLOCK = 1024


def _router_body(x_ref, w_ref, bias_ref, out_ref):
    logits = jnp.dot(x_ref[...], w_ref[...], preferred_element_type=jnp.float32)
    logits = logits + bias_ref[...]
    rowmax = jnp.max(logits, axis=-1, keepdims=True)
    work = jnp.where(logits == rowmax, -jnp.inf, logits)
    for _ in range(_K - 2):
        m = jnp.max(work, axis=-1, keepdims=True)
        work = jnp.where(work == m, -jnp.inf, work)
    thresh = jnp.max(work, axis=-1, keepdims=True)
    ew = jnp.where(logits >= thresh, jnp.exp(logits - rowmax), 0.0)
    out_ref[...] = ew / jnp.sum(ew, axis=-1, keepdims=True)


def kernel(x, w_gate, b_gate, expert_biases):
    t_dim, d_dim = x.shape
    e_dim = w_gate.shape[1]
    bias = (b_gate + expert_biases).reshape(1, e_dim).astype(jnp.float32)
    return pl.pallas_call(
        _router_body,
        grid=(t_dim // _T_BLOCK,),
        in_specs=[
            pl.BlockSpec((_T_BLOCK, d_dim), lambda i: (i, 0)),
            pl.BlockSpec((d_dim, e_dim), lambda i: (0, 0)),
            pl.BlockSpec((1, e_dim), lambda i: (0, 0)),
        ],
        out_specs=pl.BlockSpec((_T_BLOCK, e_dim), lambda i: (i, 0)),
        out_shape=jax.ShapeDtypeStruct((t_dim, e_dim), jnp.float32),
        compiler_params=pltpu.CompilerParams(
            dimension_semantics=("parallel",),
        ),
    )(x, w_gate, bias)
